# Initial kernel scaffold; baseline (speedup 1.0000x reference)
import jax
import jax.numpy as jnp
from jax.experimental import pallas as pl


def kernel(x, edge_index, W1, b1, W2, b2, W3, b3, fW1, fb1, fW2, fb2, fW3, fb3):
    raise NotImplementedError("write your pallas kernel here")


# speedup vs baseline: 7.1843x; 7.1843x over previous
import functools

import jax
import jax.numpy as jnp
from jax import lax
from jax.experimental import pallas as pl
from jax.experimental.pallas import tpu as pltpu
from jax.experimental.pallas import tpu_sc as plsc

N = 10000
D = 128
NCLS = 121
NC, NS = 2, 16
NW = NC * NS
CH = 128
NCH = 80
EPW = CH * NCH
E_PAD = NW * EPW
ACC_R = 10240
RPT = ACC_R // NS
CPT = RPT // CH
R = 1000
GRID = N // R

_mesh = plsc.VectorSubcoreMesh(core_axis_name="c", subcore_axis_name="s",
                               num_cores=NC, num_subcores=NS)


@functools.partial(
    pl.kernel,
    out_type=jax.ShapeDtypeStruct((NC, ACC_R), jnp.float32),
    mesh=_mesh,
    scratch_types=[
        pltpu.VMEM_SHARED((ACC_R,), jnp.float32),
        pltpu.VMEM((NCH, CH), jnp.int32),
        pltpu.VMEM((CH,), jnp.float32),
        pltpu.VMEM((RPT,), jnp.float32),
    ],
)
def _deg_kernel(dst_hbm, degp_hbm, dacc, dst_v, ones_v, zbuf):
    c = lax.axis_index("c")
    s = lax.axis_index("s")
    wid = c * NS + s
    for k in range(CH // 16):
        ones_v[pl.ds(k * 16, 16)] = jnp.ones((16,), jnp.float32)
    for k in range(RPT // 16):
        zbuf[pl.ds(k * 16, 16)] = jnp.zeros((16,), jnp.float32)
    pltpu.sync_copy(zbuf, dacc.at[pl.ds(s * RPT, RPT)])
    pltpu.sync_copy(dst_hbm.at[wid], dst_v)
    plsc.subcore_barrier()

    def body(j, carry):
        pltpu.sync_copy(ones_v, dacc.at[dst_v.at[j]], add=True)
        return carry

    lax.fori_loop(0, NCH, body, 0)
    plsc.subcore_barrier()
    pltpu.sync_copy(dacc.at[pl.ds(s * RPT, RPT)], zbuf)
    pltpu.sync_copy(zbuf, degp_hbm.at[c, pl.ds(s * RPT, RPT)])


GRP = 16
NGRP = NCH // GRP


@functools.partial(
    pl.kernel,
    out_type=jax.ShapeDtypeStruct((NC, ACC_R, D), jnp.float32),
    mesh=_mesh,
    scratch_types=[
        pltpu.VMEM_SHARED((ACC_R, D), jnp.float32),
        pltpu.VMEM((GRP, CH), jnp.int32),
        pltpu.VMEM((GRP, CH), jnp.int32),
        pltpu.VMEM((CH, D), jnp.float32),
        pltpu.VMEM((CH, D), jnp.float32),
        pltpu.SemaphoreType.DMA,
        pltpu.SemaphoreType.DMA,
    ],
)
def _segsum_kernel(g_hbm, src_hbm, dst_hbm, zeros_hbm, part_hbm,
                   acc, src_v, dst_v, buf_a, buf_b, sem_a, sem_b):
    c = lax.axis_index("c")
    s = lax.axis_index("s")
    wid = c * NS + s
    pltpu.sync_copy(zeros_hbm, buf_a)
    for k in range(CPT):
        pltpu.sync_copy(buf_a, acc.at[pl.ds((s * CPT + k) * CH, CH)])
    plsc.subcore_barrier()

    def body(i, carry):
        c0 = 2 * i
        pltpu.async_copy(g_hbm.at[src_v.at[c0 + 1]], buf_b, sem_b)
        pltpu.make_async_copy(g_hbm.at[src_v.at[c0]], buf_a, sem_a).wait()
        pltpu.sync_copy(buf_a, acc.at[dst_v.at[c0]], add=True)

        @pl.when(c0 + 2 < GRP)
        def _():
            pltpu.async_copy(g_hbm.at[src_v.at[c0 + 2]], buf_a, sem_a)

        pltpu.make_async_copy(g_hbm.at[src_v.at[c0 + 1]], buf_b, sem_b).wait()
        pltpu.sync_copy(buf_b, acc.at[dst_v.at[c0 + 1]], add=True)
        return carry

    for g in range(NGRP):
        pltpu.sync_copy(src_hbm.at[wid, pl.ds(g * GRP, GRP)], src_v)
        pltpu.sync_copy(dst_hbm.at[wid, pl.ds(g * GRP, GRP)], dst_v)
        pltpu.async_copy(g_hbm.at[src_v.at[0]], buf_a, sem_a)
        lax.fori_loop(0, GRP // 2, body, 0)

    plsc.subcore_barrier()
    for k in range(CPT):
        r = (s * CPT + k) * CH
        pltpu.sync_copy(acc.at[pl.ds(r, CH)], buf_a)
        pltpu.sync_copy(buf_a, part_hbm.at[c, pl.ds(r, CH)])


def _l1_body(dega_ref, degb_ref, x_ref, w_ref, o_ref):
    dinv = lax.rsqrt(dega_ref[...] + degb_ref[...] + 1.0)
    o_ref[...] = jnp.dot(x_ref[...] * dinv, w_ref[...],
                         preferred_element_type=jnp.float32)


_l1 = pl.pallas_call(
    _l1_body,
    grid=(GRID,),
    in_specs=[
        pl.BlockSpec((R, 1), lambda i: (i, 0)),
        pl.BlockSpec((R, 1), lambda i: (i, 0)),
        pl.BlockSpec((R, D), lambda i: (i, 0)),
        pl.BlockSpec((D, D), lambda i: (0, 0)),
    ],
    out_specs=pl.BlockSpec((R, D), lambda i: (i, 0)),
    out_shape=jax.ShapeDtypeStruct((N, D), jnp.float32),
)


def _comb_body(dega_ref, degb_ref, g_ref, pa_ref, pb_ref, b_ref, w_ref,
               o_ref):
    dinv = lax.rsqrt(dega_ref[...] + degb_ref[...] + 1.0)
    h = jnp.maximum(
        dinv * (g_ref[...] + pa_ref[0] + pb_ref[0]) + b_ref[...], 0.0)
    o_ref[...] = jnp.dot(h * dinv, w_ref[...],
                         preferred_element_type=jnp.float32)


_comb = pl.pallas_call(
    _comb_body,
    grid=(GRID,),
    in_specs=[
        pl.BlockSpec((R, 1), lambda i: (i, 0)),
        pl.BlockSpec((R, 1), lambda i: (i, 0)),
        pl.BlockSpec((R, D), lambda i: (i, 0)),
        pl.BlockSpec((1, R, D), lambda i: (0, i, 0)),
        pl.BlockSpec((1, R, D), lambda i: (1, i, 0)),
        pl.BlockSpec((1, D), lambda i: (0, 0)),
        pl.BlockSpec((D, D), lambda i: (0, 0)),
    ],
    out_specs=pl.BlockSpec((R, D), lambda i: (i, 0)),
    out_shape=jax.ShapeDtypeStruct((N, D), jnp.float32),
)


def _head_body(dega_ref, degb_ref, g_ref, pa_ref, pb_ref, b3_ref,
               fw1_ref, fb1_ref, fw2_ref, fb2_ref, fw3_ref, fb3_ref,
               o_ref):
    dinv = lax.rsqrt(dega_ref[...] + degb_ref[...] + 1.0)
    y = jnp.maximum(
        dinv * (g_ref[...] + pa_ref[0] + pb_ref[0]) + b3_ref[...], 0.0)
    y = jnp.maximum(
        jnp.dot(y, fw1_ref[...], preferred_element_type=jnp.float32)
        + fb1_ref[...], 0.0)
    y = jnp.maximum(
        jnp.dot(y, fw2_ref[...], preferred_element_type=jnp.float32)
        + fb2_ref[...], 0.0)
    o_ref[...] = (jnp.dot(y, fw3_ref[...], preferred_element_type=jnp.float32)
                  + fb3_ref[...])


_head = pl.pallas_call(
    _head_body,
    grid=(GRID,),
    in_specs=[
        pl.BlockSpec((R, 1), lambda i: (i, 0)),
        pl.BlockSpec((R, 1), lambda i: (i, 0)),
        pl.BlockSpec((R, D), lambda i: (i, 0)),
        pl.BlockSpec((1, R, D), lambda i: (0, i, 0)),
        pl.BlockSpec((1, R, D), lambda i: (1, i, 0)),
        pl.BlockSpec((1, D), lambda i: (0, 0)),
        pl.BlockSpec((D, D), lambda i: (0, 0)),
        pl.BlockSpec((1, D), lambda i: (0, 0)),
        pl.BlockSpec((D, D), lambda i: (0, 0)),
        pl.BlockSpec((1, D), lambda i: (0, 0)),
        pl.BlockSpec((D, D), lambda i: (0, 0)),
        pl.BlockSpec((1, D), lambda i: (0, 0)),
    ],
    out_specs=pl.BlockSpec((R, D), lambda i: (i, 0)),
    out_shape=jax.ShapeDtypeStruct((N, D), jnp.float32),
)


def kernel(x, edge_index, W1, b1, W2, b2, W3, b3, fW1, fb1, fW2, fb2,
           fW3, fb3):
    src = edge_index[0].astype(jnp.int32)
    dst = edge_index[1].astype(jnp.int32)
    e = src.shape[0]
    pad = E_PAD - e
    src3 = jnp.concatenate(
        [src, jnp.zeros((pad,), jnp.int32)]).reshape(NW, NCH, CH)
    dst3 = jnp.concatenate(
        [dst, jnp.full((pad,), N, jnp.int32)]).reshape(NW, NCH, CH)
    zeros128 = jnp.zeros((CH, D), jnp.float32)

    degp = _deg_kernel(dst3)
    dega = degp[0, :N].reshape(N, 1)
    degb = degp[1, :N].reshape(N, 1)

    b1r = b1.reshape(1, D)
    b2r = b2.reshape(1, D)
    b3r = b3.reshape(1, D)
    fb1r = fb1.reshape(1, D)
    fb2r = fb2.reshape(1, D)
    fW3p = jnp.zeros((D, D), jnp.float32).at[:, :NCLS].set(fW3)
    fb3p = jnp.zeros((1, D), jnp.float32).at[0, :NCLS].set(fb3)

    g1 = _l1(dega, degb, x, W1)
    p1 = _segsum_kernel(g1, src3, dst3, zeros128)
    g2 = _comb(dega, degb, g1, p1, p1, b1r, W2)
    p2 = _segsum_kernel(g2, src3, dst3, zeros128)
    g3 = _comb(dega, degb, g2, p2, p2, b2r, W3)
    p3 = _segsum_kernel(g3, src3, dst3, zeros128)
    y = _head(dega, degb, g3, p3, p3, b3r, fW1, fb1r, fW2, fb2r, fW3p,
              fb3p)
    return y[:, :NCLS]
